# baseline (speedup 1.0000x reference)
import jax
import jax.numpy as jnp
from jax.experimental import pallas as pl
from jax.experimental.pallas import tpu as pltpu
from jax.sharding import Mesh, NamedSharding, PartitionSpec as P_

B, N, C = 8, 576, 1024
LEFT = 72
CLUSTER_K = 32
BB = 2


def _kernel(q_ref, kcls_ref, kk_ref, x_ref, out_ref):
    f32 = jnp.float32
    i32 = jnp.int32
    bf16 = jnp.bfloat16
    scale = f32(C) ** f32(-0.5)
    ones_row = jnp.ones((1, C), dtype=f32)
    ones_n_bf = jnp.ones((1, N), dtype=bf16)
    io_c = jax.lax.broadcasted_iota(i32, (N, N), 0)
    io_r = jax.lax.broadcasted_iota(i32, (N, N), 1)
    tri = io_c < io_r
    rank_rows = jax.lax.broadcasted_iota(i32, (LEFT, N), 0).astype(f32)
    iota2 = jax.lax.broadcasted_iota(i32, (LEFT, N), 1)

    per_b = []
    skeys = []
    for bb in range(BB):
        q2 = q_ref[bb]
        kcls = kcls_ref[bb]
        kk = kk_ref[bb]

        lk = jax.lax.dot_general(q2, kk, (((1,), (1,)), ((), ())),
                                 preferred_element_type=f32)
        lcls = jnp.sum(q2 * kcls) * scale
        lk = lk * scale
        m = jnp.maximum(jnp.max(lk), lcls)
        ek = jnp.exp(lk - m)
        s = jnp.sum(ek) + jnp.exp(lcls - m)
        attn = ek / s
        attn_col = jnp.exp(jnp.transpose(lk) - m) / s

        beats = ((attn_col > attn) | ((attn_col == attn) & tri)).astype(bf16)
        rank = jax.lax.dot_general(ones_n_bf, beats, (((1,), (0,)), ((), ())),
                                   preferred_element_type=f32)
        P = (rank == rank_rows).astype(f32)
        S = (rank < f32(LEFT)).astype(f32)

        nsq = jax.lax.dot_general(ones_row, kk * kk, (((1,), (1,)), ((), ())),
                                  preferred_element_type=f32)
        invn = 1.0 / jnp.maximum(jnp.sqrt(nsq), f32(1e-12))

        ksel = jax.lax.dot_general(P, kk, (((1,), (0,)), ((), ())),
                                   preferred_element_type=f32)
        invnsel = jax.lax.dot_general(P, invn, (((1,), (1,)), ((), ())),
                                      preferred_element_type=f32)
        cos = jax.lax.dot_general(ksel, kk, (((1,), (1,)), ((), ())),
                                  preferred_element_type=f32)
        cos = cos * invnsel * invn
        cos = jnp.where(P > 0.5, f32(-3.0), cos)

        bits = jax.lax.bitcast_convert_type(cos, i32)
        skey = bits ^ (jax.lax.shift_right_arithmetic(bits, 31)
                       & i32(0x7FFFFFFF))
        skey = (skey & i32(~1023)) | (i32(1023) - iota2)
        skeys.append(skey)
        per_b.append((attn, P, S))

    neg_inf_key = i32(-(2 ** 31) + 1)

    def top32_body(i, carry):
        kw, M = carry
        mx = jnp.max(kw, axis=1, keepdims=True)
        oh = kw == mx
        return jnp.where(oh, neg_inf_key, kw), M + oh.astype(f32)

    skey_all = jnp.concatenate(skeys, axis=0)
    _, M_all = jax.lax.fori_loop(
        0, CLUSTER_K, top32_body,
        (skey_all, jnp.zeros((BB * LEFT, N), dtype=f32)), unroll=True)

    for bb in range(BB):
        attn, P, S = per_b[bb]
        M = M_all[bb * LEFT:(bb + 1) * LEFT, :]
        x = x_ref[bb]
        res = jax.lax.dot_general(P + M * attn, x, (((1,), (0,)), ((), ())),
                                  preferred_element_type=f32)
        extra = jax.lax.dot_general((1.0 - S) * attn, x,
                                    (((1,), (0,)), ((), ())),
                                    preferred_element_type=f32)
        out_ref[bb, 0:LEFT, :] = res
        out_ref[bb, LEFT:LEFT + 1, :] = extra


def _run(q0, kcls, kk, x):
    nb = q0.shape[0]
    return pl.pallas_call(
        _kernel,
        grid=(nb // BB,),
        in_specs=[
            pl.BlockSpec((BB, 1, C), lambda b: (b, 0, 0)),
            pl.BlockSpec((BB, 1, C), lambda b: (b, 0, 0)),
            pl.BlockSpec((BB, N, C), lambda b: (b, 0, 0)),
            pl.BlockSpec((BB, N, C), lambda b: (b, 0, 0)),
        ],
        out_specs=pl.BlockSpec((BB, LEFT + 1, C), lambda b: (b, 0, 0)),
        out_shape=jax.ShapeDtypeStruct((nb, LEFT + 1, C), jnp.float32),
        compiler_params=pltpu.CompilerParams(
            dimension_semantics=("arbitrary",)),
    )(q0, kcls, kk, x)


def kernel(image_features, desired_q, desired_k):
    q0 = desired_q[:, 0:1, :]
    kcls = desired_k[:, 0:1, :]
    kk = desired_k[:, 1:, :]
    devs = jax.devices()
    if len(devs) >= 2:
        mesh = Mesh(devs[:2], ("b",))
        spec = P_("b")
        fn = jax.shard_map(_run, mesh=mesh, in_specs=(spec, spec, spec, spec),
                           out_specs=spec, check_vma=False)
        return fn(q0, kcls, kk, image_features)
    return _run(q0, kcls, kk, image_features)


# speedup vs baseline: 7.2732x; 7.2732x over previous
import jax
import jax.numpy as jnp
from jax.experimental import pallas as pl
from jax.experimental.pallas import tpu as pltpu
from jax.sharding import Mesh, NamedSharding, PartitionSpec as P_

B, N, C = 8, 576, 1024
LEFT = 72
CLUSTER_K = 32
BB = 2


def _kernel(q_ref, kcls_ref, kk_ref, x_ref, out_ref):
    f32 = jnp.float32
    i32 = jnp.int32
    bf16 = jnp.bfloat16
    scale = f32(C) ** f32(-0.5)
    ones_row = jnp.ones((1, C), dtype=f32)
    ones_n_bf = jnp.ones((1, N), dtype=bf16)
    io_c = jax.lax.broadcasted_iota(i32, (N, N), 0)
    io_r = jax.lax.broadcasted_iota(i32, (N, N), 1)
    tri = io_c < io_r
    rank_rows = jax.lax.broadcasted_iota(i32, (LEFT, N), 0).astype(f32)
    iota2 = jax.lax.broadcasted_iota(i32, (LEFT, N), 1)

    per_b = []
    skeys = []
    for bb in range(BB):
        q2 = q_ref[bb]
        kcls = kcls_ref[bb]
        kk = kk_ref[bb]

        lk = jax.lax.dot_general(q2, kk, (((1,), (1,)), ((), ())),
                                 preferred_element_type=f32)
        lcls = jnp.sum(q2 * kcls) * scale
        lk = lk * scale
        m = jnp.maximum(jnp.max(lk), lcls)
        ek = jnp.exp(lk - m)
        s = jnp.sum(ek) + jnp.exp(lcls - m)
        attn = ek / s
        attn_col = jnp.exp(jnp.transpose(lk) - m) / s

        beats = ((attn_col > attn) | ((attn_col == attn) & tri)).astype(bf16)
        rank = jax.lax.dot_general(ones_n_bf, beats, (((1,), (0,)), ((), ())),
                                   preferred_element_type=f32)
        P = (rank == rank_rows).astype(f32)
        S = (rank < f32(LEFT)).astype(f32)

        nsq = jax.lax.dot_general(ones_row, kk * kk, (((1,), (1,)), ((), ())),
                                  preferred_element_type=f32)
        invn = 1.0 / jnp.maximum(jnp.sqrt(nsq), f32(1e-12))

        ksel = jax.lax.dot_general(P, kk, (((1,), (0,)), ((), ())),
                                   preferred_element_type=f32)
        invnsel = jax.lax.dot_general(P, invn, (((1,), (1,)), ((), ())),
                                      preferred_element_type=f32)
        cos = jax.lax.dot_general(ksel, kk, (((1,), (1,)), ((), ())),
                                  preferred_element_type=f32)
        cos = cos * invnsel * invn
        cos = jnp.where(P > 0.5, f32(-3.0), cos)

        bits = jax.lax.bitcast_convert_type(cos, i32)
        skey = bits ^ (jax.lax.shift_right_arithmetic(bits, 31)
                       & i32(0x7FFFFFFF))
        skey = (skey & i32(~1023)) | (i32(1023) - iota2)
        skeys.append(skey)
        per_b.append((attn, P, S))

    neg_inf_key = i32(-(2 ** 31) + 1)

    def top32_body(i, carry):
        kw, M = carry
        mx = jnp.max(kw, axis=1, keepdims=True)
        oh = kw == mx
        return jnp.where(oh, neg_inf_key, kw), M + oh.astype(f32)

    skey_all = jnp.concatenate(skeys, axis=0)
    _, M_all = jax.lax.fori_loop(
        0, CLUSTER_K, top32_body,
        (skey_all, jnp.zeros((BB * LEFT, N), dtype=f32)), unroll=True)

    for bb in range(BB):
        attn, P, S = per_b[bb]
        M = M_all[bb * LEFT:(bb + 1) * LEFT, :]
        x = x_ref[bb]
        res = jax.lax.dot_general(P + M * attn, x, (((1,), (0,)), ((), ())),
                                  preferred_element_type=f32)
        extra = jax.lax.dot_general((1.0 - S) * attn, x,
                                    (((1,), (0,)), ((), ())),
                                    preferred_element_type=f32)
        out_ref[bb, 0:LEFT, :] = res
        out_ref[bb, LEFT:LEFT + 1, :] = extra


def _run(q0, kcls, kk, x):
    nb = q0.shape[0]
    return pl.pallas_call(
        _kernel,
        grid=(nb // BB,),
        in_specs=[
            pl.BlockSpec((BB, 1, C), lambda b: (b, 0, 0)),
            pl.BlockSpec((BB, 1, C), lambda b: (b, 0, 0)),
            pl.BlockSpec((BB, N, C), lambda b: (b, 0, 0)),
            pl.BlockSpec((BB, N, C), lambda b: (b, 0, 0)),
        ],
        out_specs=pl.BlockSpec((BB, LEFT + 1, C), lambda b: (b, 0, 0)),
        out_shape=jax.ShapeDtypeStruct((nb, LEFT + 1, C), jnp.float32),
        compiler_params=pltpu.CompilerParams(
            dimension_semantics=("arbitrary",)),
    )(q0, kcls, kk, x)


def kernel(image_features, desired_q, desired_k):
    q0 = desired_q[:, 0:1, :]
    kcls = desired_k[:, 0:1, :]
    kk = desired_k[:, 1:, :]
    return _run(q0, kcls, kk, image_features)
